# Initial kernel scaffold; baseline (speedup 1.0000x reference)
import jax
import jax.numpy as jnp
from jax.experimental import pallas as pl


def kernel(tokens, dispatch_weights, combine_weights, gate_w, value_w, out_w, scales):
    raise NotImplementedError("write your pallas kernel here")


# speedup vs baseline: 2.9276x; 2.9276x over previous
import functools

import jax
import jax.numpy as jnp
from jax.experimental import pallas as pl
from jax.experimental.pallas import tpu as pltpu


def _ffn_kernel(disp_ref, cw_ref, x_ref, gw_ref, vw_ref, ow_ref, out_ref):
    e = pl.program_id(0)
    k = pl.program_id(1)

    @pl.when((e == 0) & (k == 0))
    def _init():
        out_ref[...] = jnp.zeros_like(out_ref)

    x = x_ref[...]
    gw = gw_ref[0]
    vw = vw_ref[0]
    ow = ow_ref[0]

    g = jnp.dot(x, gw.T, preferred_element_type=jnp.float32)
    v = jnp.dot(x, vw.T, preferred_element_type=jnp.float32)
    gelu = g * 0.5 * (1.0 + jax.lax.erf(g * 0.7071067811865476))
    h = gelu * v
    contrib = jnp.dot(h, ow.T, preferred_element_type=jnp.float32)

    w = jnp.where(disp_ref[0] > 0.0, cw_ref[0], 0.0)
    out_ref[...] += contrib * w


def kernel(tokens, dispatch_weights, combine_weights, gate_w, value_w, out_w, scales):
    B, N, D = tokens.shape
    E, H, _ = gate_w.shape
    x = tokens.reshape(B * N, D)
    disp = dispatch_weights.reshape(B * N, E).T.reshape(E, B * N, 1)
    cw = (combine_weights.reshape(B * N, E) * scales[None, :]).T.reshape(E, B * N, 1)

    Hc = 512
    K = H // Hc
    grid = (E, K)

    out = pl.pallas_call(
        _ffn_kernel,
        grid=grid,
        in_specs=[
            pl.BlockSpec((1, B * N, 1), lambda e, k: (e, 0, 0)),
            pl.BlockSpec((1, B * N, 1), lambda e, k: (e, 0, 0)),
            pl.BlockSpec((B * N, D), lambda e, k: (0, 0)),
            pl.BlockSpec((1, Hc, D), lambda e, k: (e, k, 0)),
            pl.BlockSpec((1, Hc, D), lambda e, k: (e, k, 0)),
            pl.BlockSpec((1, D, Hc), lambda e, k: (e, 0, k)),
        ],
        out_specs=pl.BlockSpec((B * N, D), lambda e, k: (0, 0)),
        out_shape=jax.ShapeDtypeStruct((B * N, D), jnp.float32),
        compiler_params=pltpu.CompilerParams(
            dimension_semantics=("arbitrary", "arbitrary"),
        ),
    )(disp, cw, x, gate_w, value_w, out_w)
    return out.reshape(B, N, D)


# speedup vs baseline: 2.9288x; 1.0004x over previous
import functools

import jax
import jax.numpy as jnp
from jax.experimental import pallas as pl
from jax.experimental.pallas import tpu as pltpu


def _ffn_kernel(disp_ref, cw_ref, x_ref, gw_ref, vw_ref, ow_ref, out_ref):
    e = pl.program_id(0)
    k = pl.program_id(1)

    @pl.when((e == 0) & (k == 0))
    def _init():
        out_ref[...] = jnp.zeros_like(out_ref)

    x = x_ref[...].astype(jnp.bfloat16)
    gw = gw_ref[0].astype(jnp.bfloat16)
    vw = vw_ref[0].astype(jnp.bfloat16)
    ow = ow_ref[0].astype(jnp.bfloat16)

    g = jnp.dot(x, gw.T, preferred_element_type=jnp.float32)
    v = jnp.dot(x, vw.T, preferred_element_type=jnp.float32)
    gelu = g * 0.5 * (1.0 + jax.lax.erf(g * 0.7071067811865476))
    h = (gelu * v).astype(jnp.bfloat16)
    contrib = jnp.dot(h, ow.T, preferred_element_type=jnp.float32)

    w = jnp.where(disp_ref[0] > 0.0, cw_ref[0], 0.0)
    out_ref[...] += contrib * w


def kernel(tokens, dispatch_weights, combine_weights, gate_w, value_w, out_w, scales):
    B, N, D = tokens.shape
    E, H, _ = gate_w.shape
    x = tokens.reshape(B * N, D)
    disp = dispatch_weights.reshape(B * N, E).T.reshape(E, B * N, 1)
    cw = (combine_weights.reshape(B * N, E) * scales[None, :]).T.reshape(E, B * N, 1)

    Hc = 512
    K = H // Hc
    grid = (E, K)

    out = pl.pallas_call(
        _ffn_kernel,
        grid=grid,
        in_specs=[
            pl.BlockSpec((1, B * N, 1), lambda e, k: (e, 0, 0)),
            pl.BlockSpec((1, B * N, 1), lambda e, k: (e, 0, 0)),
            pl.BlockSpec((B * N, D), lambda e, k: (0, 0)),
            pl.BlockSpec((1, Hc, D), lambda e, k: (e, k, 0)),
            pl.BlockSpec((1, Hc, D), lambda e, k: (e, k, 0)),
            pl.BlockSpec((1, D, Hc), lambda e, k: (e, 0, k)),
        ],
        out_specs=pl.BlockSpec((B * N, D), lambda e, k: (0, 0)),
        out_shape=jax.ShapeDtypeStruct((B * N, D), jnp.float32),
        compiler_params=pltpu.CompilerParams(
            dimension_semantics=("arbitrary", "arbitrary"),
        ),
    )(disp, cw, x, gate_w, value_w, out_w)
    return out.reshape(B, N, D)
